# baseline (speedup 1.0000x reference)
import functools

import jax
import jax.numpy as jnp
from jax import lax
from jax.experimental import pallas as pl
from jax.experimental.pallas import tpu as pltpu
from jax.experimental.pallas import tpu_sc as plsc

MAG = 0.5

_TOTAL = 16384 * 2048
_NW = 32
_PER_W = _TOTAL // _NW
_CH = 16384
_NCH = _PER_W // _CH
_VECS = _CH // 16


_BATCH = 8


def _sc_map_chunk(src, dst):
    half = jnp.full((16,), MAG, jnp.float32)
    two = jnp.full((16,), 1.0 / MAG, jnp.float32)
    one = jnp.full((16,), 1.0, jnp.float32)

    @pl.loop(0, _VECS // _BATCH)
    def _vec(i):
        base = i * (16 * _BATCH)
        cs = [src[pl.ds(base + k * 16, 16)] for k in range(_BATCH)]
        rs = [jnp.where(c == 1.0, half, jnp.where(c == -1.0, two, one))
              for c in cs]
        for k in range(_BATCH):
            dst[pl.ds(base + k * 16, 16)] = rs[k]


_DEPTH = 3


def _sc_body(cond_hbm, out_hbm, in0, in1, in2, ou0, ou1, ou2, sem_in, sem_out):
    wid = lax.axis_index("s") * 2 + lax.axis_index("c")
    base = wid * _PER_W
    inb = (in0, in1, in2)
    oub = (ou0, ou1, ou2)

    for g in range(_DEPTH):
        pltpu.async_copy(cond_hbm.at[pl.ds(base + g * _CH, _CH)], inb[g], sem_in)

    for g in range(_NCH):
        src, dst = inb[g % _DEPTH], oub[g % _DEPTH]
        pltpu.make_async_copy(cond_hbm.at[pl.ds(0, _CH)], src, sem_in).wait()
        if g >= _DEPTH:
            pltpu.make_async_copy(dst, out_hbm.at[pl.ds(0, _CH)], sem_out).wait()
        _sc_map_chunk(src, dst)
        pltpu.async_copy(dst, out_hbm.at[pl.ds(base + g * _CH, _CH)], sem_out)
        if g + _DEPTH < _NCH:
            pltpu.async_copy(
                cond_hbm.at[pl.ds(base + (g + _DEPTH) * _CH, _CH)], src, sem_in)

    for g in range(_DEPTH):
        pltpu.make_async_copy(oub[g], out_hbm.at[pl.ds(0, _CH)], sem_out).wait()


@jax.jit
def _sc_run(flat):
    mesh = plsc.VectorSubcoreMesh(core_axis_name="c", subcore_axis_name="s")
    return pl.kernel(
        _sc_body,
        out_type=jax.ShapeDtypeStruct((_TOTAL,), jnp.float32),
        mesh=mesh,
        scratch_types=[
            pltpu.VMEM((_CH,), jnp.float32),
            pltpu.VMEM((_CH,), jnp.float32),
            pltpu.VMEM((_CH,), jnp.float32),
            pltpu.VMEM((_CH,), jnp.float32),
            pltpu.VMEM((_CH,), jnp.float32),
            pltpu.VMEM((_CH,), jnp.float32),
            pltpu.SemaphoreType.DMA,
            pltpu.SemaphoreType.DMA,
        ],
    )(flat)


def kernel(cond):
    n, m = cond.shape
    return _sc_run(cond.reshape(-1)).reshape(n, m)


# speedup vs baseline: 4.2158x; 4.2158x over previous
import jax
import jax.numpy as jnp
from jax.experimental import pallas as pl
from jax.experimental.pallas import tpu as pltpu

MAG = 0.5

_N, _M = 16384, 2048
_CH_ROWS = 512
_NCH = _N // _CH_ROWS
_DEPTH = 4


def _map_block(c):
    stp = jnp.where(c == 1.0, jnp.float32(MAG), jnp.float32(1.0))
    return jnp.where(c == -1.0, jnp.float32(1.0 / MAG), stp)


def _stream_body(cond_hbm, out_hbm, *rest):
    inb = rest[0:_DEPTH]
    oub = rest[_DEPTH:2 * _DEPTH]
    sin = rest[2 * _DEPTH:3 * _DEPTH]
    sout = rest[3 * _DEPTH:4 * _DEPTH]

    for j in range(_DEPTH):
        pltpu.async_copy(
            cond_hbm.at[pl.ds(j * _CH_ROWS, _CH_ROWS), :], inb[j], sin[j])

    @pl.loop(0, _NCH // _DEPTH)
    def _outer(o):
        base = o * _DEPTH
        for j in range(_DEPTH):
            g = base + j
            pltpu.make_async_copy(
                cond_hbm.at[pl.ds(0, _CH_ROWS), :], inb[j], sin[j]).wait()

            @pl.when(g >= _DEPTH)
            def _():
                pltpu.make_async_copy(
                    oub[j], out_hbm.at[pl.ds(0, _CH_ROWS), :], sout[j]).wait()

            oub[j][...] = _map_block(inb[j][...])
            pltpu.async_copy(
                oub[j], out_hbm.at[pl.ds(g * _CH_ROWS, _CH_ROWS), :], sout[j])

            @pl.when(g + _DEPTH < _NCH)
            def _():
                pltpu.async_copy(
                    cond_hbm.at[pl.ds((g + _DEPTH) * _CH_ROWS, _CH_ROWS), :],
                    inb[j], sin[j])

    for j in range(_DEPTH):
        pltpu.make_async_copy(
            oub[j], out_hbm.at[pl.ds(0, _CH_ROWS), :], sout[j]).wait()


def kernel(cond):
    n, m = cond.shape
    return pl.pallas_call(
        _stream_body,
        in_specs=[pl.BlockSpec(memory_space=pltpu.HBM)],
        out_specs=pl.BlockSpec(memory_space=pltpu.HBM),
        out_shape=jax.ShapeDtypeStruct((n, m), cond.dtype),
        scratch_shapes=(
            [pltpu.VMEM((_CH_ROWS, _M), jnp.float32) for _ in range(2 * _DEPTH)]
            + [pltpu.SemaphoreType.DMA for _ in range(2 * _DEPTH)]
        ),
    )(cond)
